# baseline (speedup 1.0000x reference)
import jax
import jax.numpy as jnp
from jax import lax
from jax.experimental import pallas as pl
from jax.experimental.pallas import tpu as pltpu

_B, _C, _S, _N, _D = 8, 16, 4, 256, 16
_DLP, _DM, _NCLS = 16, 16, 2


def _vgl_body(adj_ref, feat_ref, wlp_ref, wm1_ref, wm2_ref, wdec_ref,
              bdec_ref, out_ref, h_scr, gm_scr):
    b = pl.program_id(0)
    s = pl.program_id(1)
    c = pl.program_id(2)

    fw = jnp.dot(feat_ref[0, 0, 0], wlp_ref[0, 0],
                 preferred_element_type=jnp.float32)
    h = jnp.maximum(
        jnp.dot(adj_ref[0, 0, 0], fw, preferred_element_type=jnp.float32),
        0.0)
    h_scr[c] = h

    @pl.when(c == _C - 1)
    def _end_of_section():
        hm = h_scr[...].reshape(_C, _N * _DLP)
        gm = lax.dot_general(hm, hm, (((1,), (1,)), ((), ())),
                             preferred_element_type=jnp.float32)

        @pl.when(s == 0)
        def _():
            gm_scr[...] = gm

        @pl.when(s > 0)
        def _():
            gm_scr[...] += gm

        @pl.when(s == _S - 1)
        def _end_of_batch():
            g = gm_scr[...]
            rows = lax.broadcasted_iota(jnp.int32, (_C, _C), 0)
            cols = lax.broadcasted_iota(jnp.int32, (_C, _C), 1)
            eye = (rows == cols).astype(jnp.float32)
            dcol = jnp.sum(g * eye, axis=1, keepdims=True)
            drow = jnp.sum(g * eye, axis=0, keepdims=True)
            denom = (jnp.sqrt(dcol) + 1e-8) * (jnp.sqrt(drow) + 1e-8)
            bg = g / denom
            h1 = jnp.maximum(
                jnp.dot(bg, wm1_ref[...], preferred_element_type=jnp.float32),
                0.0)
            h2 = jnp.maximum(
                jnp.dot(bg, jnp.dot(h1, wm2_ref[...],
                                    preferred_element_type=jnp.float32),
                        preferred_element_type=jnp.float32),
                0.0)
            dec = jnp.dot(h2, wdec_ref[...],
                          preferred_element_type=jnp.float32) + bdec_ref[...]
            pooled = jnp.mean(dec, axis=0, keepdims=True)
            out_ref[pl.ds(b, 1), :] = jax.nn.sigmoid(pooled)


def kernel(feats, adjs, W_lp, W_m1, W_m2, W_dec, b_dec):
    b_dec2 = b_dec.reshape(1, _NCLS)
    grid = (_B, _S, _C)
    return pl.pallas_call(
        _vgl_body,
        grid=grid,
        in_specs=[
            pl.BlockSpec((1, 1, 1, _N, _N), lambda b, s, c: (b, c, s, 0, 0)),
            pl.BlockSpec((1, 1, 1, _N, _D), lambda b, s, c: (b, c, s, 0, 0)),
            pl.BlockSpec((1, 1, _D, _DLP), lambda b, s, c: (c, s, 0, 0)),
            pl.BlockSpec((_C, _DM), lambda b, s, c: (0, 0)),
            pl.BlockSpec((_DM, _DM), lambda b, s, c: (0, 0)),
            pl.BlockSpec((_DM, _NCLS), lambda b, s, c: (0, 0)),
            pl.BlockSpec((1, _NCLS), lambda b, s, c: (0, 0)),
        ],
        out_specs=pl.BlockSpec((_B, _NCLS), lambda b, s, c: (0, 0)),
        out_shape=jax.ShapeDtypeStruct((_B, _NCLS), jnp.float32),
        scratch_shapes=[
            pltpu.VMEM((_C, _N, _DLP), jnp.float32),
            pltpu.VMEM((_C, _C), jnp.float32),
        ],
    )(adjs, feats, W_lp, W_m1, W_m2, W_dec, b_dec2)


# speedup vs baseline: 2.3870x; 2.3870x over previous
import jax
import jax.numpy as jnp
from jax import lax
from jax.experimental import pallas as pl
from jax.experimental.pallas import tpu as pltpu

_B, _C, _S, _N, _D = 8, 16, 4, 256, 16
_DLP, _DM, _NCLS = 16, 16, 2


def _vgl_body(adj_ref, feat_ref, wlp_ref, wm1_ref, wm2_ref, wdec_ref,
              bdec_ref, out_ref, z_scr):
    b = pl.program_id(0)
    c = pl.program_id(1)

    for s in range(_S):
        fwT = lax.dot_general(wlp_ref[0, s], feat_ref[0, 0, s],
                              (((0,), (1,)), ((), ())),
                              preferred_element_type=jnp.float32)
        hT = jnp.maximum(
            lax.dot_general(fwT, adj_ref[0, 0, s],
                            (((1,), (1,)), ((), ())),
                            preferred_element_type=jnp.float32),
            0.0)
        z_scr[pl.ds(c, 1), pl.ds(s * _DLP, _DLP), :] = hT[None]

    @pl.when(c == _C - 1)
    def _end_of_batch():
        z = z_scr[...].reshape(_C, _S * _DLP * _N)
        g = lax.dot_general(z, z, (((1,), (1,)), ((), ())),
                            preferred_element_type=jnp.float32)
        rows = lax.broadcasted_iota(jnp.int32, (_C, _C), 0)
        cols = lax.broadcasted_iota(jnp.int32, (_C, _C), 1)
        eye = (rows == cols).astype(jnp.float32)
        dcol = jnp.sum(g * eye, axis=1, keepdims=True)
        drow = jnp.sum(g * eye, axis=0, keepdims=True)
        denom = (jnp.sqrt(dcol) + 1e-8) * (jnp.sqrt(drow) + 1e-8)
        bg = g / denom
        h1 = jnp.maximum(
            jnp.dot(bg, wm1_ref[...], preferred_element_type=jnp.float32),
            0.0)
        h2 = jnp.maximum(
            jnp.dot(bg, jnp.dot(h1, wm2_ref[...],
                                preferred_element_type=jnp.float32),
                    preferred_element_type=jnp.float32),
            0.0)
        dec = jnp.dot(h2, wdec_ref[...],
                      preferred_element_type=jnp.float32) + bdec_ref[...]
        pooled = jnp.mean(dec, axis=0, keepdims=True)
        out_ref[pl.ds(b, 1), :] = jax.nn.sigmoid(pooled)


def kernel(feats, adjs, W_lp, W_m1, W_m2, W_dec, b_dec):
    b_dec2 = b_dec.reshape(1, _NCLS)
    grid = (_B, _C)
    return pl.pallas_call(
        _vgl_body,
        grid=grid,
        in_specs=[
            pl.BlockSpec((1, 1, _S, _N, _N), lambda b, c: (b, c, 0, 0, 0)),
            pl.BlockSpec((1, 1, _S, _N, _D), lambda b, c: (b, c, 0, 0, 0)),
            pl.BlockSpec((1, _S, _D, _DLP), lambda b, c: (c, 0, 0, 0)),
            pl.BlockSpec((_C, _DM), lambda b, c: (0, 0)),
            pl.BlockSpec((_DM, _DM), lambda b, c: (0, 0)),
            pl.BlockSpec((_DM, _NCLS), lambda b, c: (0, 0)),
            pl.BlockSpec((1, _NCLS), lambda b, c: (0, 0)),
        ],
        out_specs=pl.BlockSpec((_B, _NCLS), lambda b, c: (0, 0)),
        out_shape=jax.ShapeDtypeStruct((_B, _NCLS), jnp.float32),
        scratch_shapes=[
            pltpu.VMEM((_C, _S * _DLP, _N), jnp.float32),
        ],
    )(adjs, feats, W_lp, W_m1, W_m2, W_dec, b_dec2)


# speedup vs baseline: 3.7694x; 1.5791x over previous
import jax
import jax.numpy as jnp
from jax import lax
from jax.experimental import pallas as pl
from jax.experimental.pallas import tpu as pltpu

_B, _C, _S, _N, _D = 8, 16, 4, 256, 16
_DLP, _DM, _NCLS = 16, 16, 2
_CPB = 8


def _vgl_body(adj_ref, feat_ref, wlp_ref, wm1_ref, wm2_ref, wdec_ref,
              bdec_ref, out_ref, z_scr):
    b = pl.program_id(0)
    cb = pl.program_id(1)

    for cc in range(_CPB):
        for s in range(_S):
            fwT = lax.dot_general(wlp_ref[cc, s], feat_ref[0, cc, s],
                                  (((0,), (1,)), ((), ())),
                                  preferred_element_type=jnp.float32)
            hT = jnp.maximum(
                lax.dot_general(fwT, adj_ref[0, cc, s],
                                (((1,), (1,)), ((), ())),
                                preferred_element_type=jnp.float32),
                0.0)
            z_scr[pl.ds(cb * _CPB + cc, 1), pl.ds(s * _DLP, _DLP), :] = hT[None]

    @pl.when(cb == (_C // _CPB) - 1)
    def _end_of_batch():
        z = z_scr[...].reshape(_C, _S * _DLP * _N)
        g = lax.dot_general(z, z, (((1,), (1,)), ((), ())),
                            preferred_element_type=jnp.float32)
        rows = lax.broadcasted_iota(jnp.int32, (_C, _C), 0)
        cols = lax.broadcasted_iota(jnp.int32, (_C, _C), 1)
        eye = (rows == cols).astype(jnp.float32)
        dcol = jnp.sum(g * eye, axis=1, keepdims=True)
        drow = jnp.sum(g * eye, axis=0, keepdims=True)
        denom = (jnp.sqrt(dcol) + 1e-8) * (jnp.sqrt(drow) + 1e-8)
        bg = g / denom
        h1 = jnp.maximum(
            jnp.dot(bg, wm1_ref[...], preferred_element_type=jnp.float32),
            0.0)
        h2 = jnp.maximum(
            jnp.dot(bg, jnp.dot(h1, wm2_ref[...],
                                preferred_element_type=jnp.float32),
                    preferred_element_type=jnp.float32),
            0.0)
        dec = jnp.dot(h2, wdec_ref[...],
                      preferred_element_type=jnp.float32) + bdec_ref[...]
        pooled = jnp.mean(dec, axis=0, keepdims=True)
        out_ref[pl.ds(b, 1), :] = jax.nn.sigmoid(pooled)


def kernel(feats, adjs, W_lp, W_m1, W_m2, W_dec, b_dec):
    b_dec2 = b_dec.reshape(1, _NCLS)
    grid = (_B, _C // _CPB)
    return pl.pallas_call(
        _vgl_body,
        grid=grid,
        in_specs=[
            pl.BlockSpec((1, _CPB, _S, _N, _N), lambda b, c: (b, c, 0, 0, 0)),
            pl.BlockSpec((1, _CPB, _S, _N, _D), lambda b, c: (b, c, 0, 0, 0)),
            pl.BlockSpec((_CPB, _S, _D, _DLP), lambda b, c: (c, 0, 0, 0)),
            pl.BlockSpec((_C, _DM), lambda b, c: (0, 0)),
            pl.BlockSpec((_DM, _DM), lambda b, c: (0, 0)),
            pl.BlockSpec((_DM, _NCLS), lambda b, c: (0, 0)),
            pl.BlockSpec((1, _NCLS), lambda b, c: (0, 0)),
        ],
        out_specs=pl.BlockSpec((_B, _NCLS), lambda b, c: (0, 0)),
        out_shape=jax.ShapeDtypeStruct((_B, _NCLS), jnp.float32),
        scratch_shapes=[
            pltpu.VMEM((_C, _S * _DLP, _N), jnp.float32),
        ],
    )(adjs, feats, W_lp, W_m1, W_m2, W_dec, b_dec2)


# speedup vs baseline: 3.9788x; 1.0556x over previous
import jax
import jax.numpy as jnp
from jax import lax
from jax.experimental import pallas as pl
from jax.experimental.pallas import tpu as pltpu

_B, _C, _S, _N, _D = 8, 16, 4, 256, 16
_DLP, _DM, _NCLS = 16, 16, 2
_CPB = 16


def _vgl_body(adj_ref, feat_ref, wlp_ref, wm1_ref, wm2_ref, wdec_ref,
              bdec_ref, out_ref, z_scr):
    b = pl.program_id(0)
    cb = pl.program_id(1)

    for cc in range(_CPB):
        for s in range(_S):
            fwT = lax.dot_general(wlp_ref[cc, s], feat_ref[0, cc, s],
                                  (((0,), (1,)), ((), ())),
                                  preferred_element_type=jnp.float32)
            hT = jnp.maximum(
                lax.dot_general(fwT, adj_ref[0, cc, s],
                                (((1,), (1,)), ((), ())),
                                preferred_element_type=jnp.float32),
                0.0)
            z_scr[pl.ds(cb * _CPB + cc, 1), pl.ds(s * _DLP, _DLP), :] = hT[None]

    @pl.when(cb == (_C // _CPB) - 1)
    def _end_of_batch():
        z = z_scr[...].reshape(_C, _S * _DLP * _N)
        g = lax.dot_general(z, z, (((1,), (1,)), ((), ())),
                            preferred_element_type=jnp.float32)
        rows = lax.broadcasted_iota(jnp.int32, (_C, _C), 0)
        cols = lax.broadcasted_iota(jnp.int32, (_C, _C), 1)
        eye = (rows == cols).astype(jnp.float32)
        dcol = jnp.sum(g * eye, axis=1, keepdims=True)
        drow = jnp.sum(g * eye, axis=0, keepdims=True)
        denom = (jnp.sqrt(dcol) + 1e-8) * (jnp.sqrt(drow) + 1e-8)
        bg = g / denom
        h1 = jnp.maximum(
            jnp.dot(bg, wm1_ref[...], preferred_element_type=jnp.float32),
            0.0)
        h2 = jnp.maximum(
            jnp.dot(bg, jnp.dot(h1, wm2_ref[...],
                                preferred_element_type=jnp.float32),
                    preferred_element_type=jnp.float32),
            0.0)
        dec = jnp.dot(h2, wdec_ref[...],
                      preferred_element_type=jnp.float32) + bdec_ref[...]
        pooled = jnp.mean(dec, axis=0, keepdims=True)
        out_ref[pl.ds(b, 1), :] = jax.nn.sigmoid(pooled)


def kernel(feats, adjs, W_lp, W_m1, W_m2, W_dec, b_dec):
    b_dec2 = b_dec.reshape(1, _NCLS)
    grid = (_B, _C // _CPB)
    return pl.pallas_call(
        _vgl_body,
        grid=grid,
        in_specs=[
            pl.BlockSpec((1, _CPB, _S, _N, _N), lambda b, c: (b, c, 0, 0, 0)),
            pl.BlockSpec((1, _CPB, _S, _N, _D), lambda b, c: (b, c, 0, 0, 0)),
            pl.BlockSpec((_CPB, _S, _D, _DLP), lambda b, c: (c, 0, 0, 0)),
            pl.BlockSpec((_C, _DM), lambda b, c: (0, 0)),
            pl.BlockSpec((_DM, _DM), lambda b, c: (0, 0)),
            pl.BlockSpec((_DM, _NCLS), lambda b, c: (0, 0)),
            pl.BlockSpec((1, _NCLS), lambda b, c: (0, 0)),
        ],
        out_specs=pl.BlockSpec((_B, _NCLS), lambda b, c: (0, 0)),
        out_shape=jax.ShapeDtypeStruct((_B, _NCLS), jnp.float32),
        scratch_shapes=[
            pltpu.VMEM((_C, _S * _DLP, _N), jnp.float32),
        ],
    )(adjs, feats, W_lp, W_m1, W_m2, W_dec, b_dec2)
